# baseline (speedup 1.0000x reference)
import functools

import jax
import jax.numpy as jnp
from jax import lax
from jax.experimental import pallas as pl
from jax.experimental.pallas import tpu as pltpu
from jax.experimental.pallas import tpu_sc as plsc

_B, _H, _V = 4096, 20, 1000
_NC, _NS = 2, 16
_NW = _NC * _NS
_BT = 256
_NBP = _B // _BT
_HG = _H // (_NW // _NBP)
_PASSES = ((0, 336), (336, 336), (672, 328))


def _scatter_pass(buf, idx_ref, h, lo, sz, value):
    val = jnp.full((16,), value, jnp.float32)
    lane = lax.iota(jnp.int32, 16)
    lane_h = lax.mul(lane, jnp.full((16,), _H, jnp.int32))
    lov = jnp.full((16,), lo, jnp.int32)
    hiv = jnp.full((16,), lo + sz, jnp.int32)
    hv = jnp.full((16,), h, jnp.int32)
    for k in range(_BT // 16):
        gidx = lax.add(lane_h, lax.add(hv, jnp.full((16,), (16 * k) * _H, jnp.int32)))
        cv = plsc.load_gather(idx_ref, [gidx])
        m = jnp.logical_and(cv >= lov, cv < hiv)
        row = lax.sub(cv, lov)
        blocal = lax.add(lane, jnp.full((16,), k * 16, jnp.int32))
        plsc.store_scatter(buf, [row, blocal], val, mask=m)


def _sc_body(xf_hbm, z_hbm, o_hbm, idxs, buf, sems):
    wid = lax.axis_index("s") * _NC + lax.axis_index("c")
    hg = wid // _NBP
    b0 = (wid % _NBP) * _BT
    h_base = hg * _HG

    x_copy = pltpu.make_async_copy(
        xf_hbm.at[pl.ds(b0 * _H, _BT * _H)], idxs, sems.at[1]
    )
    x_copy.start()
    pltpu.sync_copy(z_hbm, buf)
    x_copy.wait()

    for hh in range(_HG):
        h = lax.add(h_base, hh)
        for lo, sz in _PASSES:
            _scatter_pass(buf, idxs, h, lo, sz, 1.0)
            dst = o_hbm.at[h, pl.ds(lo, sz), pl.ds(b0, _BT)]
            src = buf.at[pl.ds(0, sz)]
            pltpu.make_async_copy(src, dst, sems.at[0]).start()
            pltpu.make_async_copy(src, dst, sems.at[0]).wait()
            _scatter_pass(buf, idxs, h, lo, sz, 0.0)


@functools.partial(
    pl.kernel,
    out_type=jax.ShapeDtypeStruct((_H, _V, _B), jnp.float32),
    mesh=plsc.VectorSubcoreMesh(core_axis_name="c", subcore_axis_name="s"),
    compiler_params=pltpu.CompilerParams(
        use_tc_tiling_on_sc=True, needs_layout_passes=False
    ),
    scratch_types=[
        pltpu.VMEM((_BT * _H,), jnp.int32),
        pltpu.VMEM((336, _BT), jnp.float32),
        pltpu.SemaphoreType.DMA((2,)),
    ],
)
def _sc_onehot(xf_hbm, z_hbm, o_hbm, idxs, buf, sems):
    _sc_body(xf_hbm, z_hbm, o_hbm, idxs, buf, sems)


def kernel(x, W):
    xf = x.reshape(-1)
    z = jnp.zeros((336, _BT), jnp.float32)
    out = _sc_onehot(xf, z)
    return jnp.transpose(out, (2, 0, 1))


# speedup vs baseline: 1.0496x; 1.0496x over previous
import functools

import jax
import jax.numpy as jnp
from jax import lax
from jax.experimental import pallas as pl
from jax.experimental.pallas import tpu as pltpu
from jax.experimental.pallas import tpu_sc as plsc

_B, _H, _V = 4096, 20, 1000
_NC, _NS = 2, 16
_NW = _NC * _NS
_BT = _B // _NW


def _scatter_strip(buf, idx_ref, h, value):
    val = jnp.full((16,), value, jnp.float32)
    lane = lax.iota(jnp.int32, 16)
    lane_h = lax.mul(lane, jnp.full((16,), _H, jnp.int32))
    for k in range(_BT // 16):
        gidx = lax.add(lane_h, jnp.full((16,), (16 * k) * _H + h, jnp.int32))
        cv = plsc.load_gather(idx_ref, [gidx])
        blocal = lax.add(lane, jnp.full((16,), k * 16, jnp.int32))
        plsc.store_scatter(buf, [cv, blocal], val)


def _sc_body(xf_hbm, z_hbm, o_hbm, idxs, buf, sems):
    w = lax.axis_index("s") * _NC + lax.axis_index("c")
    b0 = w * _BT

    x_copy = pltpu.make_async_copy(
        xf_hbm.at[pl.ds(b0 * _H, _BT * _H)], idxs, sems.at[1]
    )
    x_copy.start()
    pltpu.sync_copy(z_hbm, buf)
    x_copy.wait()

    for h in range(_H):
        _scatter_strip(buf, idxs, h, 1.0)
        dst = o_hbm.at[h, :, pl.ds(b0, _BT)]
        pltpu.make_async_copy(buf, dst, sems.at[0]).start()
        pltpu.make_async_copy(buf, dst, sems.at[0]).wait()
        _scatter_strip(buf, idxs, h, 0.0)


@functools.partial(
    pl.kernel,
    out_type=jax.ShapeDtypeStruct((_H, _V, _B), jnp.float32),
    mesh=plsc.VectorSubcoreMesh(core_axis_name="c", subcore_axis_name="s"),
    compiler_params=pltpu.CompilerParams(
        use_tc_tiling_on_sc=True, needs_layout_passes=False
    ),
    scratch_types=[
        pltpu.VMEM((_BT * _H,), jnp.int32),
        pltpu.VMEM((_V, _BT), jnp.float32),
        pltpu.SemaphoreType.DMA((2,)),
    ],
)
def _sc_onehot(xf_hbm, z_hbm, o_hbm, idxs, buf, sems):
    _sc_body(xf_hbm, z_hbm, o_hbm, idxs, buf, sems)


def kernel(x, W):
    xf = x.reshape(-1)
    z = jnp.zeros((_V, _BT), jnp.float32)
    out = _sc_onehot(xf, z)
    return jnp.transpose(out, (2, 0, 1))
